# baseline (speedup 1.0000x reference)
import functools

import jax
import jax.numpy as jnp
from jax import lax
from jax.experimental import pallas as pl
from jax.experimental.pallas import tpu as pltpu
from jax.experimental.pallas import tpu_sc as plsc

KNN = 10
KP = 16
NCAND = 12
BN_EPS = 1e-5
def _dot(x, w):
  return jax.lax.dot_general(
      x, w, (((1,), (0,)), ((), ())),
      preferred_element_type=jnp.float32)


def _dotT(x, w):
  return jax.lax.dot_general(
      x, w, (((1,), (1,)), ((), ())),
      preferred_element_type=jnp.float32)


def _knn_body(q_ref, pt_ref, idx_ref):
  q = q_ref[0]
  pt = pt_ref[0]
  b = pl.program_id(0)
  n = pt.shape[1]
  p2 = jnp.sum(pt * pt, axis=0, keepdims=True)
  q2 = jnp.sum(q * q, axis=1, keepdims=True)
  d2 = q2 + p2 - 2.0 * _dot(q, pt)
  bits = jax.lax.bitcast_convert_type(d2, jnp.int32)
  skey = bits ^ ((bits >> 31) & jnp.int32(0x7FFFFFFF))
  lane = jax.lax.broadcasted_iota(jnp.int32, d2.shape, 1)
  keys = (skey & jnp.int32(~0x1FFF)) | lane
  imax = jnp.int32(0x7FFFFFFF)
  cols = []
  for _ in range(NCAND):
    kmin = jnp.min(keys, axis=1, keepdims=True)
    keys = jnp.where(keys == kmin, imax, keys)
    cols.append(kmin & jnp.int32(0x1FFF))
  cols += [jnp.zeros_like(cols[0])] * (KP - NCAND)
  idx_ref[0] = jnp.concatenate(cols, axis=1) + b * n


def _knn_topk(q_rows, p_t):
  B, M, _ = q_rows.shape
  N = p_t.shape[2]
  TM = 1024
  return pl.pallas_call(
      _knn_body,
      grid=(B, M // TM),
      in_specs=[
          pl.BlockSpec((1, TM, 3), lambda b, i: (b, i, 0)),
          pl.BlockSpec((1, 3, N), lambda b, i: (b, 0, 0)),
      ],
      out_specs=pl.BlockSpec((1, TM, KP), lambda b, i: (b, i, 0)),
      out_shape=jax.ShapeDtypeStruct((B, M, KP), jnp.int32),
  )(q_rows, p_t)


def _gather_sc(px, py, pz, idxg):
  BN = px.shape[0]
  R = idxg.shape[0]
  info = plsc.get_sparse_core_info()
  nw = info.num_cores * info.num_subcores
  per_w = R // nw
  mesh = plsc.VectorSubcoreMesh(core_axis_name="c", subcore_axis_name="s")
  fvec = jax.ShapeDtypeStruct((R,), jnp.float32)

  @functools.partial(
      pl.kernel, mesh=mesh,
      out_type=[fvec, fvec, fvec],
      compiler_params=pltpu.CompilerParams(needs_layout_passes=False),
      scratch_types=[
          pltpu.VMEM((BN,), jnp.float32),
          pltpu.VMEM((BN,), jnp.float32),
          pltpu.VMEM((BN,), jnp.float32),
          pltpu.VMEM((per_w,), jnp.int32),
          pltpu.VMEM((per_w,), jnp.float32),
          pltpu.VMEM((per_w,), jnp.float32),
          pltpu.VMEM((per_w,), jnp.float32),
      ])
  def gk(px_hbm, py_hbm, pz_hbm, idx_hbm, outx, outy, outz,
         tabx, taby, tabz, idx_v, ox, oy, oz):
    wid = lax.axis_index("s") * info.num_cores + lax.axis_index("c")
    base = wid * per_w
    pltpu.sync_copy(px_hbm, tabx)
    pltpu.sync_copy(py_hbm, taby)
    pltpu.sync_copy(pz_hbm, tabz)
    pltpu.sync_copy(idx_hbm.at[pl.ds(base, per_w)], idx_v)

    def body(i, carry):
      sl = pl.ds(i * 16, 16)
      rid = idx_v[sl]
      ox[sl] = plsc.load_gather(tabx, [rid])
      oy[sl] = plsc.load_gather(taby, [rid])
      oz[sl] = plsc.load_gather(tabz, [rid])
      return carry

    lax.fori_loop(0, per_w // 16, body, 0)
    pltpu.sync_copy(ox, outx.at[pl.ds(base, per_w)])
    pltpu.sync_copy(oy, outy.at[pl.ds(base, per_w)])
    pltpu.sync_copy(oz, outz.at[pl.ds(base, per_w)])

  return gk(px, py, pz, idxg)


def _acc_stats(st_ref, y, maskf, step):
  ym = y * maskf
  s = jnp.sum(ym, axis=0, keepdims=True)
  sq = jnp.sum(ym * ym, axis=0, keepdims=True)
  st = jnp.concatenate([s, sq], axis=0)

  @pl.when(step == 0)
  def _():
    st_ref[...] = jnp.zeros_like(st_ref)

  st_ref[...] += st


def _bf16(x):
  return x.astype(jnp.bfloat16).astype(jnp.float32)


def _rank_body(px_ref, py_ref, pz_ref, q_ref, mask_ref):
  qx = q_ref[:, 0:1]
  qy = q_ref[:, 1:2]
  qz = q_ref[:, 2:3]
  px, py, pz = px_ref[...], py_ref[...], pz_ref[...]
  q2 = qx * qx + qy * qy + qz * qz
  p2 = px * px + py * py + pz * pz
  cross = (_bf16(qx) * _bf16(px) + _bf16(qy) * _bf16(py)
           + _bf16(qz) * _bf16(pz))
  d2 = q2 + p2 - 2.0 * cross
  bits = jax.lax.bitcast_convert_type(d2, jnp.int32)
  skey = bits ^ ((bits >> 31) & jnp.int32(0x7FFFFFFF))
  pos = jax.lax.broadcasted_iota(jnp.int32, d2.shape, 1)
  imax = jnp.int32(0x7FFFFFFF)
  keys = jnp.where(pos < NCAND, (skey & jnp.int32(~0xF)) | pos, imax)
  keep = jnp.zeros(keys.shape, jnp.bool_)
  for _ in range(KNN):
    kmin = jnp.min(keys, axis=1, keepdims=True)
    sel = keys == kmin
    keep = keep | sel
    keys = jnp.where(sel, imax, keys)
  mask_ref[...] = keep.astype(jnp.float32)


def _rank_topk(pxbm, pybm, pzbm, q_flat):
  BM = pxbm.shape[0]
  TG = 1024
  return pl.pallas_call(
      _rank_body,
      grid=(BM // TG,),
      in_specs=[
          pl.BlockSpec((TG, KP), lambda i: (i, 0)),
          pl.BlockSpec((TG, KP), lambda i: (i, 0)),
          pl.BlockSpec((TG, KP), lambda i: (i, 0)),
          pl.BlockSpec((TG, 3), lambda i: (i, 0)),
      ],
      out_specs=pl.BlockSpec((TG, KP), lambda i: (i, 0)),
      out_shape=jax.ShapeDtypeStruct((BM, KP), jnp.float32),
  )(pxbm, pybm, pzbm, q_flat)


def _q_expand(q_blk, tr):
  g = tr // KP
  return jnp.broadcast_to(q_blk[:, None, :], (g, KP, 3)).reshape(tr, 3)


def _stageA_body(knn_ref, q_ref, mask_ref, w0_ref, b0_ref,
                 loc_ref, z_ref, st_ref):
  tr = knn_ref.shape[0]
  qb = _q_expand(q_ref[...], tr)
  local = qb - knn_ref[...]
  loc_ref[...] = local
  w0 = w0_ref[...]
  y = _dotT(local, w0[:, 0:3]) + _dotT(qb, w0[:, 3:6]) + b0_ref[...]
  z_ref[...] = y
  _acc_stats(st_ref, y, mask_ref[...], pl.program_id(0))


def _stageA(knn_rows, q_rows_flat, maskf, w0, b0):
  R = knn_rows.shape[0]
  TR = 4096
  TQ = TR // KP
  co = w0.shape[0]
  return pl.pallas_call(
      _stageA_body,
      grid=(R // TR,),
      in_specs=[
          pl.BlockSpec((TR, 3), lambda i: (i, 0)),
          pl.BlockSpec((TQ, 3), lambda i: (i, 0)),
          pl.BlockSpec((TR, 1), lambda i: (i, 0)),
          pl.BlockSpec(w0.shape, lambda i: (0, 0)),
          pl.BlockSpec((co,), lambda i: (0,)),
      ],
      out_specs=[
          pl.BlockSpec((TR, 3), lambda i: (i, 0)),
          pl.BlockSpec((TR, co), lambda i: (i, 0)),
          pl.BlockSpec((2, co), lambda i: (0, 0)),
      ],
      out_shape=[
          jax.ShapeDtypeStruct((R, 3), jnp.float32),
          jax.ShapeDtypeStruct((R, co), jnp.float32),
          jax.ShapeDtypeStruct((2, co), jnp.float32),
      ],
  )(knn_rows, q_rows_flat, maskf, w0, b0)


def _bn_aff(st_ref, g_ref, beta_ref, n_real):
  st = st_ref[...]
  mean = st[0:1, :] / n_real
  var = st[1:2, :] / n_real - mean * mean
  a = g_ref[...] * jax.lax.rsqrt(var + BN_EPS)
  c = beta_ref[...] - mean * a
  return a, c


def _bn_mm_body(z_ref, st_in_ref, g_ref, beta_ref, w_ref, b_ref, mask_ref,
                out_ref, st_ref, *, n_real):
  a, c = _bn_aff(st_in_ref, g_ref, beta_ref, n_real)
  x = z_ref[...] * a + c
  x = jnp.where(x >= 0, x, 0.2 * x)
  y = _dotT(x, w_ref[...]) + b_ref[...]
  out_ref[...] = y
  _acc_stats(st_ref, y, mask_ref[...], pl.program_id(0))


def _bn_mm(z, st_in, g, beta, w, b, maskf, n_real):
  R, ci = z.shape
  co = w.shape[0]
  TR = 8192
  body = functools.partial(_bn_mm_body, n_real=n_real)
  out_specs = [pl.BlockSpec((TR, co), lambda i: (i, 0)),
               pl.BlockSpec((2, co), lambda i: (0, 0))]
  out_shape = [jax.ShapeDtypeStruct((R, co), jnp.float32),
               jax.ShapeDtypeStruct((2, co), jnp.float32)]
  return pl.pallas_call(
      body,
      grid=(R // TR,),
      in_specs=[
          pl.BlockSpec((TR, ci), lambda i: (i, 0)),
          pl.BlockSpec((2, ci), lambda i: (0, 0)),
          pl.BlockSpec((ci,), lambda i: (0,)),
          pl.BlockSpec((ci,), lambda i: (0,)),
          pl.BlockSpec((co, ci), lambda i: (0, 0)),
          pl.BlockSpec((co,), lambda i: (0,)),
          pl.BlockSpec((TR, 1), lambda i: (i, 0)),
      ],
      out_specs=out_specs,
      out_shape=out_shape,
  )(z, st_in, g, beta, w, b, maskf)


def _stageD_body(z3_ref, st3_ref, g3_ref, beta3_ref, w3_ref, b3_ref,
                 loc_ref, q_ref, mask_ref, wa_ref, b0_ref,
                 out_ref, st_ref, *, n_real):
  tr = z3_ref.shape[0]
  a, c = _bn_aff(st3_ref, g3_ref, beta3_ref, n_real)
  x3 = z3_ref[...] * a + c
  x3 = jnp.where(x3 >= 0, x3, 0.2 * x3)
  feat = _dotT(x3, w3_ref[...]) + b3_ref[...]
  maskf = mask_ref[...]
  fm = jnp.where(maskf > 0.5, feat, -jnp.inf)
  g = tr // KP
  fg = jnp.max(fm.reshape(g, KP, feat.shape[1]), axis=1)
  pf = jnp.broadcast_to(fg[:, None, :], (g, KP, feat.shape[1]))
  pf = pf.reshape(tr, feat.shape[1])
  local = loc_ref[...]
  qb = _q_expand(q_ref[...], tr)
  kd = jnp.sqrt(jnp.sum(local * local, axis=1, keepdims=True))
  wa = wa_ref[...]
  y = (_dotT(local, wa[:, 0:3]) + _dotT(qb, wa[:, 3:6])
       + _dotT(kd, wa[:, 6:7]) + _dotT(feat, wa[:, 7:135])
       + _dotT(pf, wa[:, 135:263]) + b0_ref[...])
  out_ref[...] = y
  _acc_stats(st_ref, y, maskf, pl.program_id(0))


def _stageD(z3, st3, g3, beta3, w3, b3, local, q_rows_flat, maskf,
            wa, b0, n_real):
  R, ci = z3.shape
  co = wa.shape[0]
  TR = 4096
  TQ = TR // KP
  body = functools.partial(_stageD_body, n_real=n_real)
  return pl.pallas_call(
      body,
      grid=(R // TR,),
      in_specs=[
          pl.BlockSpec((TR, ci), lambda i: (i, 0)),
          pl.BlockSpec((2, ci), lambda i: (0, 0)),
          pl.BlockSpec((ci,), lambda i: (0,)),
          pl.BlockSpec((ci,), lambda i: (0,)),
          pl.BlockSpec(w3.shape, lambda i: (0, 0)),
          pl.BlockSpec((w3.shape[0],), lambda i: (0,)),
          pl.BlockSpec((TR, 3), lambda i: (i, 0)),
          pl.BlockSpec((TQ, 3), lambda i: (i, 0)),
          pl.BlockSpec((TR, 1), lambda i: (i, 0)),
          pl.BlockSpec(wa.shape, lambda i: (0, 0)),
          pl.BlockSpec((co,), lambda i: (0,)),
      ],
      out_specs=[
          pl.BlockSpec((TR, co), lambda i: (i, 0)),
          pl.BlockSpec((2, co), lambda i: (0, 0)),
      ],
      out_shape=[
          jax.ShapeDtypeStruct((R, co), jnp.float32),
          jax.ShapeDtypeStruct((2, co), jnp.float32),
      ],
  )(z3, st3, g3, beta3, w3, b3, local, q_rows_flat, maskf, wa, b0)


def _stageG_body(z_ref, st_in_ref, g_ref, beta_ref, w_ref, b_ref, loc_ref,
                 mask_ref, out_ref, *, n_real):
  tr = z_ref.shape[0]
  a, c = _bn_aff(st_in_ref, g_ref, beta_ref, n_real)
  x = z_ref[...] * a + c
  x = jnp.where(x >= 0, x, 0.2 * x)
  wlog = _dot(x, w_ref[...]) + b_ref[...]
  g = tr // KP
  w3 = wlog.reshape(g, KP, 1)
  mask = mask_ref[...].reshape(g, KP, 1) > 0.5
  mx = jnp.max(jnp.where(mask, w3, -jnp.inf), axis=1, keepdims=True)
  e = jnp.where(mask, jnp.exp(w3 - mx), 0.0)
  s = jnp.sum(e, axis=1, keepdims=True)
  w = e / s
  loc3 = loc_ref[...].reshape(g, KP, 3)
  vec = jnp.sum(w * loc3, axis=1)
  out_ref[...] = jnp.sqrt(jnp.sum(vec * vec, axis=1, keepdims=True))


def _stageG(z, st_in, g, beta, w, b, local, maskf, n_real):
  R, ci = z.shape
  TR = 4096
  G = TR // KP
  body = functools.partial(_stageG_body, n_real=n_real)
  return pl.pallas_call(
      body,
      grid=(R // TR,),
      in_specs=[
          pl.BlockSpec((TR, ci), lambda i: (i, 0)),
          pl.BlockSpec((2, ci), lambda i: (0, 0)),
          pl.BlockSpec((ci,), lambda i: (0,)),
          pl.BlockSpec((ci,), lambda i: (0,)),
          pl.BlockSpec((ci, 1), lambda i: (0, 0)),
          pl.BlockSpec((1, 1), lambda i: (0, 0)),
          pl.BlockSpec((TR, 3), lambda i: (i, 0)),
          pl.BlockSpec((TR, 1), lambda i: (i, 0)),
      ],
      out_specs=pl.BlockSpec((G, 1), lambda i: (i, 0)),
      out_shape=jax.ShapeDtypeStruct((R // KP, 1), jnp.float32),
  )(z, st_in, g, beta, w, b, local, maskf)


def kernel(input_pcd, query_points, params):
  B, N, _ = input_pcd.shape
  M = query_points.shape[2]
  R = B * M * KP
  n_real = float(B * M * KNN)

  q_rows = jnp.transpose(query_points, (0, 2, 1))
  p_t = jnp.transpose(input_pcd, (0, 2, 1))
  q_flat = q_rows.reshape(B * M, 3)

  idx = _knn_topk(q_rows, p_t)
  idxg = idx.reshape(R)

  pf = input_pcd.reshape(B * N, 3)
  gx, gy, gz = _gather_sc(pf[:, 0], pf[:, 1], pf[:, 2], idxg)
  knn_rows = jnp.stack([gx, gy, gz], axis=1)

  maskbm = _rank_topk(gx.reshape(B * M, KP), gy.reshape(B * M, KP),
                      gz.reshape(B * M, KP), q_flat)
  maskf = maskbm.reshape(R, 1)

  p = params
  local, z1, st1 = _stageA(knn_rows, q_flat, maskf, p['patch_W0'],
                           p['patch_b0'])

  z2, st2 = _bn_mm(z1, st1, p['patch_g0'], p['patch_beta0'],
                   p['patch_W1'], p['patch_b1'], maskf, n_real)
  z3, st3 = _bn_mm(z2, st2, p['patch_g1'], p['patch_beta1'],
                   p['patch_W2'], p['patch_b2'], maskf, n_real)

  z1a, st1a = _stageD(
      z3, st3, p['patch_g2'], p['patch_beta2'],
      p['patch_W3'], p['patch_b3'], local, q_flat, maskf,
      p['attn_W0'], p['attn_b0'], n_real)

  z2a, st2a = _bn_mm(z1a, st1a, p['attn_g0'], p['attn_beta0'],
                     p['attn_W1'], p['attn_b1'], maskf, n_real)
  z3a, st3a = _bn_mm(z2a, st2a, p['attn_g1'], p['attn_beta1'],
                     p['attn_W2'], p['attn_b2'], maskf, n_real)

  udf = _stageG(z3a, st3a, p['attn_g2'], p['attn_beta2'],
                jnp.transpose(p['attn_W3']), p['attn_b3'][None, :],
                local, maskf, n_real)
  return udf.reshape(B, M)


# speedup vs baseline: 1.0519x; 1.0519x over previous
import functools

import jax
import jax.numpy as jnp
from jax import lax
from jax.experimental import pallas as pl
from jax.experimental.pallas import tpu as pltpu
from jax.experimental.pallas import tpu_sc as plsc

KNN = 10
KP = 16
NCAND = 12
BN_EPS = 1e-5
def _dot(x, w):
  return jax.lax.dot_general(
      x, w, (((1,), (0,)), ((), ())),
      preferred_element_type=jnp.float32)


def _dotT(x, w):
  return jax.lax.dot_general(
      x, w, (((1,), (1,)), ((), ())),
      preferred_element_type=jnp.float32)


def _knn_body(q_ref, pt_ref, idx_ref):
  q = q_ref[0]
  pt = pt_ref[0]
  b = pl.program_id(0)
  n = pt.shape[1]
  p2 = jnp.sum(pt * pt, axis=0, keepdims=True)
  q2 = jnp.sum(q * q, axis=1, keepdims=True)
  d2 = q2 + p2 - 2.0 * _dot(q, pt)
  bits = jax.lax.bitcast_convert_type(d2, jnp.int32)
  skey = bits ^ ((bits >> 31) & jnp.int32(0x7FFFFFFF))
  lane = jax.lax.broadcasted_iota(jnp.int32, d2.shape, 1)
  keys = (skey & jnp.int32(~0x1FFF)) | lane
  imax = jnp.int32(0x7FFFFFFF)
  cols = []
  for _ in range(NCAND):
    kmin = jnp.min(keys, axis=1, keepdims=True)
    keys = jnp.where(keys == kmin, imax, keys)
    cols.append(kmin & jnp.int32(0x1FFF))
  cols += [jnp.zeros_like(cols[0])] * (KP - NCAND)
  idx_ref[0] = jnp.concatenate(cols, axis=1) + b * n


def _knn_topk(q_rows, p_t):
  B, M, _ = q_rows.shape
  N = p_t.shape[2]
  TM = 512
  return pl.pallas_call(
      _knn_body,
      grid=(B, M // TM),
      in_specs=[
          pl.BlockSpec((1, TM, 3), lambda b, i: (b, i, 0)),
          pl.BlockSpec((1, 3, N), lambda b, i: (b, 0, 0)),
      ],
      out_specs=pl.BlockSpec((1, TM, KP), lambda b, i: (b, i, 0)),
      out_shape=jax.ShapeDtypeStruct((B, M, KP), jnp.int32),
  )(q_rows, p_t)


def _gather_sc(px, py, pz, idxg):
  BN = px.shape[0]
  R = idxg.shape[0]
  info = plsc.get_sparse_core_info()
  nw = info.num_cores * info.num_subcores
  per_w = R // nw
  mesh = plsc.VectorSubcoreMesh(core_axis_name="c", subcore_axis_name="s")
  fvec = jax.ShapeDtypeStruct((R,), jnp.float32)

  @functools.partial(
      pl.kernel, mesh=mesh,
      out_type=[fvec, fvec, fvec],
      compiler_params=pltpu.CompilerParams(needs_layout_passes=False),
      scratch_types=[
          pltpu.VMEM((BN,), jnp.float32),
          pltpu.VMEM((BN,), jnp.float32),
          pltpu.VMEM((BN,), jnp.float32),
          pltpu.VMEM((per_w,), jnp.int32),
          pltpu.VMEM((per_w,), jnp.float32),
          pltpu.VMEM((per_w,), jnp.float32),
          pltpu.VMEM((per_w,), jnp.float32),
      ])
  def gk(px_hbm, py_hbm, pz_hbm, idx_hbm, outx, outy, outz,
         tabx, taby, tabz, idx_v, ox, oy, oz):
    wid = lax.axis_index("s") * info.num_cores + lax.axis_index("c")
    base = wid * per_w
    pltpu.sync_copy(px_hbm, tabx)
    pltpu.sync_copy(py_hbm, taby)
    pltpu.sync_copy(pz_hbm, tabz)
    pltpu.sync_copy(idx_hbm.at[pl.ds(base, per_w)], idx_v)

    def body(i, carry):
      sl = pl.ds(i * 16, 16)
      rid = idx_v[sl]
      ox[sl] = plsc.load_gather(tabx, [rid])
      oy[sl] = plsc.load_gather(taby, [rid])
      oz[sl] = plsc.load_gather(tabz, [rid])
      return carry

    lax.fori_loop(0, per_w // 16, body, 0)
    pltpu.sync_copy(ox, outx.at[pl.ds(base, per_w)])
    pltpu.sync_copy(oy, outy.at[pl.ds(base, per_w)])
    pltpu.sync_copy(oz, outz.at[pl.ds(base, per_w)])

  return gk(px, py, pz, idxg)


def _acc_stats(st_ref, y, maskf, step):
  ym = y * maskf
  s = jnp.sum(ym, axis=0, keepdims=True)
  sq = jnp.sum(ym * ym, axis=0, keepdims=True)
  st = jnp.concatenate([s, sq], axis=0)

  @pl.when(step == 0)
  def _():
    st_ref[...] = jnp.zeros_like(st_ref)

  st_ref[...] += st


def _bf16(x):
  return x.astype(jnp.bfloat16).astype(jnp.float32)


def _rank_body(px_ref, py_ref, pz_ref, q_ref, mask_ref):
  qx = q_ref[:, 0:1]
  qy = q_ref[:, 1:2]
  qz = q_ref[:, 2:3]
  px, py, pz = px_ref[...], py_ref[...], pz_ref[...]
  q2 = qx * qx + qy * qy + qz * qz
  p2 = px * px + py * py + pz * pz
  cross = (_bf16(qx) * _bf16(px) + _bf16(qy) * _bf16(py)
           + _bf16(qz) * _bf16(pz))
  d2 = q2 + p2 - 2.0 * cross
  bits = jax.lax.bitcast_convert_type(d2, jnp.int32)
  skey = bits ^ ((bits >> 31) & jnp.int32(0x7FFFFFFF))
  pos = jax.lax.broadcasted_iota(jnp.int32, d2.shape, 1)
  imax = jnp.int32(0x7FFFFFFF)
  keys = jnp.where(pos < NCAND, (skey & jnp.int32(~0xF)) | pos, imax)
  keep = jnp.zeros(keys.shape, jnp.bool_)
  for _ in range(KNN):
    kmin = jnp.min(keys, axis=1, keepdims=True)
    sel = keys == kmin
    keep = keep | sel
    keys = jnp.where(sel, imax, keys)
  mask_ref[...] = keep.astype(jnp.float32)


def _rank_topk(pxbm, pybm, pzbm, q_flat):
  BM = pxbm.shape[0]
  TG = 1024
  return pl.pallas_call(
      _rank_body,
      grid=(BM // TG,),
      in_specs=[
          pl.BlockSpec((TG, KP), lambda i: (i, 0)),
          pl.BlockSpec((TG, KP), lambda i: (i, 0)),
          pl.BlockSpec((TG, KP), lambda i: (i, 0)),
          pl.BlockSpec((TG, 3), lambda i: (i, 0)),
      ],
      out_specs=pl.BlockSpec((TG, KP), lambda i: (i, 0)),
      out_shape=jax.ShapeDtypeStruct((BM, KP), jnp.float32),
  )(pxbm, pybm, pzbm, q_flat)


def _q_expand(q_blk, tr):
  g = tr // KP
  return jnp.broadcast_to(q_blk[:, None, :], (g, KP, 3)).reshape(tr, 3)


def _stageA_body(knn_ref, q_ref, mask_ref, w0_ref, b0_ref,
                 loc_ref, z_ref, st_ref):
  tr = knn_ref.shape[0]
  qb = _q_expand(q_ref[...], tr)
  local = qb - knn_ref[...]
  loc_ref[...] = local
  w0 = w0_ref[...]
  y = _dotT(local, w0[:, 0:3]) + _dotT(qb, w0[:, 3:6]) + b0_ref[...]
  z_ref[...] = y
  _acc_stats(st_ref, y, mask_ref[...], pl.program_id(0))


def _stageA(knn_rows, q_rows_flat, maskf, w0, b0):
  R = knn_rows.shape[0]
  TR = 4096
  TQ = TR // KP
  co = w0.shape[0]
  return pl.pallas_call(
      _stageA_body,
      grid=(R // TR,),
      in_specs=[
          pl.BlockSpec((TR, 3), lambda i: (i, 0)),
          pl.BlockSpec((TQ, 3), lambda i: (i, 0)),
          pl.BlockSpec((TR, 1), lambda i: (i, 0)),
          pl.BlockSpec(w0.shape, lambda i: (0, 0)),
          pl.BlockSpec((co,), lambda i: (0,)),
      ],
      out_specs=[
          pl.BlockSpec((TR, 3), lambda i: (i, 0)),
          pl.BlockSpec((TR, co), lambda i: (i, 0)),
          pl.BlockSpec((2, co), lambda i: (0, 0)),
      ],
      out_shape=[
          jax.ShapeDtypeStruct((R, 3), jnp.float32),
          jax.ShapeDtypeStruct((R, co), jnp.float32),
          jax.ShapeDtypeStruct((2, co), jnp.float32),
      ],
  )(knn_rows, q_rows_flat, maskf, w0, b0)


def _bn_aff(st_ref, g_ref, beta_ref, n_real):
  st = st_ref[...]
  mean = st[0:1, :] / n_real
  var = st[1:2, :] / n_real - mean * mean
  a = g_ref[...] * jax.lax.rsqrt(var + BN_EPS)
  c = beta_ref[...] - mean * a
  return a, c


def _bn_mm_body(z_ref, st_in_ref, g_ref, beta_ref, w_ref, b_ref, mask_ref,
                out_ref, st_ref, *, n_real):
  a, c = _bn_aff(st_in_ref, g_ref, beta_ref, n_real)
  x = z_ref[...] * a + c
  x = jnp.where(x >= 0, x, 0.2 * x)
  y = _dotT(x, w_ref[...]) + b_ref[...]
  out_ref[...] = y
  _acc_stats(st_ref, y, mask_ref[...], pl.program_id(0))


def _bn_mm(z, st_in, g, beta, w, b, maskf, n_real):
  R, ci = z.shape
  co = w.shape[0]
  TR = 8192
  body = functools.partial(_bn_mm_body, n_real=n_real)
  out_specs = [pl.BlockSpec((TR, co), lambda i: (i, 0)),
               pl.BlockSpec((2, co), lambda i: (0, 0))]
  out_shape = [jax.ShapeDtypeStruct((R, co), jnp.float32),
               jax.ShapeDtypeStruct((2, co), jnp.float32)]
  return pl.pallas_call(
      body,
      grid=(R // TR,),
      in_specs=[
          pl.BlockSpec((TR, ci), lambda i: (i, 0)),
          pl.BlockSpec((2, ci), lambda i: (0, 0)),
          pl.BlockSpec((ci,), lambda i: (0,)),
          pl.BlockSpec((ci,), lambda i: (0,)),
          pl.BlockSpec((co, ci), lambda i: (0, 0)),
          pl.BlockSpec((co,), lambda i: (0,)),
          pl.BlockSpec((TR, 1), lambda i: (i, 0)),
      ],
      out_specs=out_specs,
      out_shape=out_shape,
  )(z, st_in, g, beta, w, b, maskf)


def _stageD_body(z3_ref, st3_ref, g3_ref, beta3_ref, w3_ref, b3_ref,
                 loc_ref, q_ref, mask_ref, wa_ref, b0_ref,
                 out_ref, st_ref, *, n_real):
  tr = z3_ref.shape[0]
  a, c = _bn_aff(st3_ref, g3_ref, beta3_ref, n_real)
  x3 = z3_ref[...] * a + c
  x3 = jnp.where(x3 >= 0, x3, 0.2 * x3)
  feat = _dotT(x3, w3_ref[...]) + b3_ref[...]
  maskf = mask_ref[...]
  fm = jnp.where(maskf > 0.5, feat, -jnp.inf)
  g = tr // KP
  fg = jnp.max(fm.reshape(g, KP, feat.shape[1]), axis=1)
  pf = jnp.broadcast_to(fg[:, None, :], (g, KP, feat.shape[1]))
  pf = pf.reshape(tr, feat.shape[1])
  local = loc_ref[...]
  qb = _q_expand(q_ref[...], tr)
  kd = jnp.sqrt(jnp.sum(local * local, axis=1, keepdims=True))
  wa = wa_ref[...]
  y = (_dotT(local, wa[:, 0:3]) + _dotT(qb, wa[:, 3:6])
       + _dotT(kd, wa[:, 6:7]) + _dotT(feat, wa[:, 7:135])
       + _dotT(pf, wa[:, 135:263]) + b0_ref[...])
  out_ref[...] = y
  _acc_stats(st_ref, y, maskf, pl.program_id(0))


def _stageD(z3, st3, g3, beta3, w3, b3, local, q_rows_flat, maskf,
            wa, b0, n_real):
  R, ci = z3.shape
  co = wa.shape[0]
  TR = 4096
  TQ = TR // KP
  body = functools.partial(_stageD_body, n_real=n_real)
  return pl.pallas_call(
      body,
      grid=(R // TR,),
      in_specs=[
          pl.BlockSpec((TR, ci), lambda i: (i, 0)),
          pl.BlockSpec((2, ci), lambda i: (0, 0)),
          pl.BlockSpec((ci,), lambda i: (0,)),
          pl.BlockSpec((ci,), lambda i: (0,)),
          pl.BlockSpec(w3.shape, lambda i: (0, 0)),
          pl.BlockSpec((w3.shape[0],), lambda i: (0,)),
          pl.BlockSpec((TR, 3), lambda i: (i, 0)),
          pl.BlockSpec((TQ, 3), lambda i: (i, 0)),
          pl.BlockSpec((TR, 1), lambda i: (i, 0)),
          pl.BlockSpec(wa.shape, lambda i: (0, 0)),
          pl.BlockSpec((co,), lambda i: (0,)),
      ],
      out_specs=[
          pl.BlockSpec((TR, co), lambda i: (i, 0)),
          pl.BlockSpec((2, co), lambda i: (0, 0)),
      ],
      out_shape=[
          jax.ShapeDtypeStruct((R, co), jnp.float32),
          jax.ShapeDtypeStruct((2, co), jnp.float32),
      ],
  )(z3, st3, g3, beta3, w3, b3, local, q_rows_flat, maskf, wa, b0)


def _stageG_body(z_ref, st_in_ref, g_ref, beta_ref, w_ref, b_ref, loc_ref,
                 mask_ref, out_ref, *, n_real):
  tr = z_ref.shape[0]
  a, c = _bn_aff(st_in_ref, g_ref, beta_ref, n_real)
  x = z_ref[...] * a + c
  x = jnp.where(x >= 0, x, 0.2 * x)
  wlog = _dot(x, w_ref[...]) + b_ref[...]
  g = tr // KP
  w3 = wlog.reshape(g, KP, 1)
  mask = mask_ref[...].reshape(g, KP, 1) > 0.5
  mx = jnp.max(jnp.where(mask, w3, -jnp.inf), axis=1, keepdims=True)
  e = jnp.where(mask, jnp.exp(w3 - mx), 0.0)
  s = jnp.sum(e, axis=1, keepdims=True)
  w = e / s
  loc3 = loc_ref[...].reshape(g, KP, 3)
  vec = jnp.sum(w * loc3, axis=1)
  out_ref[...] = jnp.sqrt(jnp.sum(vec * vec, axis=1, keepdims=True))


def _stageG(z, st_in, g, beta, w, b, local, maskf, n_real):
  R, ci = z.shape
  TR = 4096
  G = TR // KP
  body = functools.partial(_stageG_body, n_real=n_real)
  return pl.pallas_call(
      body,
      grid=(R // TR,),
      in_specs=[
          pl.BlockSpec((TR, ci), lambda i: (i, 0)),
          pl.BlockSpec((2, ci), lambda i: (0, 0)),
          pl.BlockSpec((ci,), lambda i: (0,)),
          pl.BlockSpec((ci,), lambda i: (0,)),
          pl.BlockSpec((ci, 1), lambda i: (0, 0)),
          pl.BlockSpec((1, 1), lambda i: (0, 0)),
          pl.BlockSpec((TR, 3), lambda i: (i, 0)),
          pl.BlockSpec((TR, 1), lambda i: (i, 0)),
      ],
      out_specs=pl.BlockSpec((G, 1), lambda i: (i, 0)),
      out_shape=jax.ShapeDtypeStruct((R // KP, 1), jnp.float32),
  )(z, st_in, g, beta, w, b, local, maskf)


def kernel(input_pcd, query_points, params):
  B, N, _ = input_pcd.shape
  M = query_points.shape[2]
  R = B * M * KP
  n_real = float(B * M * KNN)

  q_rows = jnp.transpose(query_points, (0, 2, 1))
  p_t = jnp.transpose(input_pcd, (0, 2, 1))
  q_flat = q_rows.reshape(B * M, 3)

  idx = _knn_topk(q_rows, p_t)
  idxg = idx.reshape(R)

  pf = input_pcd.reshape(B * N, 3)
  gx, gy, gz = _gather_sc(pf[:, 0], pf[:, 1], pf[:, 2], idxg)
  knn_rows = jnp.stack([gx, gy, gz], axis=1)

  maskbm = _rank_topk(gx.reshape(B * M, KP), gy.reshape(B * M, KP),
                      gz.reshape(B * M, KP), q_flat)
  maskf = maskbm.reshape(R, 1)

  p = params
  local, z1, st1 = _stageA(knn_rows, q_flat, maskf, p['patch_W0'],
                           p['patch_b0'])

  z2, st2 = _bn_mm(z1, st1, p['patch_g0'], p['patch_beta0'],
                   p['patch_W1'], p['patch_b1'], maskf, n_real)
  z3, st3 = _bn_mm(z2, st2, p['patch_g1'], p['patch_beta1'],
                   p['patch_W2'], p['patch_b2'], maskf, n_real)

  z1a, st1a = _stageD(
      z3, st3, p['patch_g2'], p['patch_beta2'],
      p['patch_W3'], p['patch_b3'], local, q_flat, maskf,
      p['attn_W0'], p['attn_b0'], n_real)

  z2a, st2a = _bn_mm(z1a, st1a, p['attn_g0'], p['attn_beta0'],
                     p['attn_W1'], p['attn_b1'], maskf, n_real)
  z3a, st3a = _bn_mm(z2a, st2a, p['attn_g1'], p['attn_beta1'],
                     p['attn_W2'], p['attn_b2'], maskf, n_real)

  udf = _stageG(z3a, st3a, p['attn_g2'], p['attn_beta2'],
                jnp.transpose(p['attn_W3']), p['attn_b3'][None, :],
                local, maskf, n_real)
  return udf.reshape(B, M)


# speedup vs baseline: 1.0635x; 1.0110x over previous
import functools

import jax
import jax.numpy as jnp
from jax import lax
from jax.experimental import pallas as pl
from jax.experimental.pallas import tpu as pltpu
from jax.experimental.pallas import tpu_sc as plsc

KNN = 10
KP = 16
NCAND = 12
BN_EPS = 1e-5
def _dot(x, w):
  return jax.lax.dot_general(
      x, w, (((1,), (0,)), ((), ())),
      preferred_element_type=jnp.float32)


def _dotT(x, w):
  return jax.lax.dot_general(
      x, w, (((1,), (1,)), ((), ())),
      preferred_element_type=jnp.float32)


def _knn_body(q_ref, pt_ref, idx_ref):
  q = q_ref[0]
  pt = pt_ref[0]
  b = pl.program_id(0)
  n = pt.shape[1]
  p2 = jnp.sum(pt * pt, axis=0, keepdims=True)
  q2 = jnp.sum(q * q, axis=1, keepdims=True)
  d2 = q2 + p2 - 2.0 * _dot(q, pt)
  bits = jax.lax.bitcast_convert_type(d2, jnp.int32)
  skey = bits ^ ((bits >> 31) & jnp.int32(0x7FFFFFFF))
  lane = jax.lax.broadcasted_iota(jnp.int32, d2.shape, 1)
  keys = (skey & jnp.int32(~0x1FFF)) | lane
  imax = jnp.int32(0x7FFFFFFF)
  cols = []
  for _ in range(NCAND):
    kmin = jnp.min(keys, axis=1, keepdims=True)
    keys = jnp.where(keys == kmin, imax, keys)
    cols.append(kmin & jnp.int32(0x1FFF))
  cols += [jnp.zeros_like(cols[0])] * (KP - NCAND)
  idx_ref[0] = jnp.concatenate(cols, axis=1) + b * n


def _knn_topk(q_rows, p_t):
  B, M, _ = q_rows.shape
  N = p_t.shape[2]
  TM = 512
  return pl.pallas_call(
      _knn_body,
      grid=(B, M // TM),
      in_specs=[
          pl.BlockSpec((1, TM, 3), lambda b, i: (b, i, 0)),
          pl.BlockSpec((1, 3, N), lambda b, i: (b, 0, 0)),
      ],
      out_specs=pl.BlockSpec((1, TM, KP), lambda b, i: (b, i, 0)),
      out_shape=jax.ShapeDtypeStruct((B, M, KP), jnp.int32),
  )(q_rows, p_t)


def _gather_sc(px, py, pz, idxg):
  BN = px.shape[0]
  R = idxg.shape[0]
  info = plsc.get_sparse_core_info()
  nw = info.num_cores * info.num_subcores
  per_w = R // nw
  mesh = plsc.VectorSubcoreMesh(core_axis_name="c", subcore_axis_name="s")
  fvec = jax.ShapeDtypeStruct((R,), jnp.float32)

  @functools.partial(
      pl.kernel, mesh=mesh,
      out_type=[fvec, fvec, fvec],
      compiler_params=pltpu.CompilerParams(needs_layout_passes=False),
      scratch_types=[
          pltpu.VMEM((BN,), jnp.float32),
          pltpu.VMEM((BN,), jnp.float32),
          pltpu.VMEM((BN,), jnp.float32),
          pltpu.VMEM((per_w,), jnp.int32),
          pltpu.VMEM((per_w,), jnp.float32),
          pltpu.VMEM((per_w,), jnp.float32),
          pltpu.VMEM((per_w,), jnp.float32),
      ])
  def gk(px_hbm, py_hbm, pz_hbm, idx_hbm, outx, outy, outz,
         tabx, taby, tabz, idx_v, ox, oy, oz):
    wid = lax.axis_index("s") * info.num_cores + lax.axis_index("c")
    base = wid * per_w
    pltpu.sync_copy(px_hbm, tabx)
    pltpu.sync_copy(py_hbm, taby)
    pltpu.sync_copy(pz_hbm, tabz)
    pltpu.sync_copy(idx_hbm.at[pl.ds(base, per_w)], idx_v)

    def body(i, carry):
      sl = pl.ds(i * 16, 16)
      rid = idx_v[sl]
      ox[sl] = plsc.load_gather(tabx, [rid])
      oy[sl] = plsc.load_gather(taby, [rid])
      oz[sl] = plsc.load_gather(tabz, [rid])
      return carry

    lax.fori_loop(0, per_w // 16, body, 0)
    pltpu.sync_copy(ox, outx.at[pl.ds(base, per_w)])
    pltpu.sync_copy(oy, outy.at[pl.ds(base, per_w)])
    pltpu.sync_copy(oz, outz.at[pl.ds(base, per_w)])

  return gk(px, py, pz, idxg)


def _acc_stats(st_ref, y, maskf, step):
  ym = y * maskf
  s = jnp.sum(ym, axis=0, keepdims=True)
  sq = jnp.sum(ym * ym, axis=0, keepdims=True)
  st = jnp.concatenate([s, sq], axis=0)

  @pl.when(step == 0)
  def _():
    st_ref[...] = jnp.zeros_like(st_ref)

  st_ref[...] += st


def _bf16(x):
  return x.astype(jnp.bfloat16).astype(jnp.float32)


def _rank_body(px_ref, py_ref, pz_ref, q_ref, mask_ref):
  qx = q_ref[:, 0:1]
  qy = q_ref[:, 1:2]
  qz = q_ref[:, 2:3]
  px, py, pz = px_ref[...], py_ref[...], pz_ref[...]
  q2 = qx * qx + qy * qy + qz * qz
  p2 = px * px + py * py + pz * pz
  cross = (_bf16(qx) * _bf16(px) + _bf16(qy) * _bf16(py)
           + _bf16(qz) * _bf16(pz))
  d2 = q2 + p2 - 2.0 * cross
  bits = jax.lax.bitcast_convert_type(d2, jnp.int32)
  skey = bits ^ ((bits >> 31) & jnp.int32(0x7FFFFFFF))
  pos = jax.lax.broadcasted_iota(jnp.int32, d2.shape, 1)
  imax = jnp.int32(0x7FFFFFFF)
  keys = jnp.where(pos < NCAND, (skey & jnp.int32(~0xF)) | pos, imax)
  keep = jnp.zeros(keys.shape, jnp.bool_)
  for _ in range(KNN):
    kmin = jnp.min(keys, axis=1, keepdims=True)
    sel = keys == kmin
    keep = keep | sel
    keys = jnp.where(sel, imax, keys)
  mask_ref[...] = keep.astype(jnp.float32)


def _rank_topk(pxbm, pybm, pzbm, q_flat):
  BM = pxbm.shape[0]
  TG = 1024
  return pl.pallas_call(
      _rank_body,
      grid=(BM // TG,),
      in_specs=[
          pl.BlockSpec((TG, KP), lambda i: (i, 0)),
          pl.BlockSpec((TG, KP), lambda i: (i, 0)),
          pl.BlockSpec((TG, KP), lambda i: (i, 0)),
          pl.BlockSpec((TG, 3), lambda i: (i, 0)),
      ],
      out_specs=pl.BlockSpec((TG, KP), lambda i: (i, 0)),
      out_shape=jax.ShapeDtypeStruct((BM, KP), jnp.float32),
  )(pxbm, pybm, pzbm, q_flat)


def _q_expand(q_blk, tr):
  g = tr // KP
  return jnp.broadcast_to(q_blk[:, None, :], (g, KP, 3)).reshape(tr, 3)


def _stageA_body(knn_ref, q_ref, mask_ref, w0_ref, b0_ref,
                 loc_ref, z_ref, st_ref):
  tr = knn_ref.shape[0]
  qb = _q_expand(q_ref[...], tr)
  local = qb - knn_ref[...]
  loc_ref[...] = local
  w0 = w0_ref[...]
  y = _dotT(local, w0[:, 0:3]) + _dotT(qb, w0[:, 3:6]) + b0_ref[...]
  z_ref[...] = y
  _acc_stats(st_ref, y, mask_ref[...], pl.program_id(0))


def _stageA(knn_rows, q_rows_flat, maskf, w0, b0):
  R = knn_rows.shape[0]
  TR = 8192
  TQ = TR // KP
  co = w0.shape[0]
  return pl.pallas_call(
      _stageA_body,
      grid=(R // TR,),
      in_specs=[
          pl.BlockSpec((TR, 3), lambda i: (i, 0)),
          pl.BlockSpec((TQ, 3), lambda i: (i, 0)),
          pl.BlockSpec((TR, 1), lambda i: (i, 0)),
          pl.BlockSpec(w0.shape, lambda i: (0, 0)),
          pl.BlockSpec((co,), lambda i: (0,)),
      ],
      out_specs=[
          pl.BlockSpec((TR, 3), lambda i: (i, 0)),
          pl.BlockSpec((TR, co), lambda i: (i, 0)),
          pl.BlockSpec((2, co), lambda i: (0, 0)),
      ],
      out_shape=[
          jax.ShapeDtypeStruct((R, 3), jnp.float32),
          jax.ShapeDtypeStruct((R, co), jnp.float32),
          jax.ShapeDtypeStruct((2, co), jnp.float32),
      ],
  )(knn_rows, q_rows_flat, maskf, w0, b0)


def _bn_aff(st_ref, g_ref, beta_ref, n_real):
  st = st_ref[...]
  mean = st[0:1, :] / n_real
  var = st[1:2, :] / n_real - mean * mean
  a = g_ref[...] * jax.lax.rsqrt(var + BN_EPS)
  c = beta_ref[...] - mean * a
  return a, c


def _bn_mm_body(z_ref, st_in_ref, g_ref, beta_ref, w_ref, b_ref, mask_ref,
                out_ref, st_ref, *, n_real):
  a, c = _bn_aff(st_in_ref, g_ref, beta_ref, n_real)
  x = z_ref[...] * a + c
  x = jnp.where(x >= 0, x, 0.2 * x)
  y = _dotT(x, w_ref[...]) + b_ref[...]
  out_ref[...] = y
  _acc_stats(st_ref, y, mask_ref[...], pl.program_id(0))


def _bn_mm(z, st_in, g, beta, w, b, maskf, n_real):
  R, ci = z.shape
  co = w.shape[0]
  TR = 8192
  body = functools.partial(_bn_mm_body, n_real=n_real)
  out_specs = [pl.BlockSpec((TR, co), lambda i: (i, 0)),
               pl.BlockSpec((2, co), lambda i: (0, 0))]
  out_shape = [jax.ShapeDtypeStruct((R, co), jnp.float32),
               jax.ShapeDtypeStruct((2, co), jnp.float32)]
  return pl.pallas_call(
      body,
      grid=(R // TR,),
      in_specs=[
          pl.BlockSpec((TR, ci), lambda i: (i, 0)),
          pl.BlockSpec((2, ci), lambda i: (0, 0)),
          pl.BlockSpec((ci,), lambda i: (0,)),
          pl.BlockSpec((ci,), lambda i: (0,)),
          pl.BlockSpec((co, ci), lambda i: (0, 0)),
          pl.BlockSpec((co,), lambda i: (0,)),
          pl.BlockSpec((TR, 1), lambda i: (i, 0)),
      ],
      out_specs=out_specs,
      out_shape=out_shape,
  )(z, st_in, g, beta, w, b, maskf)


def _stageD_body(z3_ref, st3_ref, g3_ref, beta3_ref, w3_ref, b3_ref,
                 loc_ref, q_ref, mask_ref, wa_ref, b0_ref,
                 out_ref, st_ref, *, n_real):
  tr = z3_ref.shape[0]
  a, c = _bn_aff(st3_ref, g3_ref, beta3_ref, n_real)
  x3 = z3_ref[...] * a + c
  x3 = jnp.where(x3 >= 0, x3, 0.2 * x3)
  feat = _dotT(x3, w3_ref[...]) + b3_ref[...]
  maskf = mask_ref[...]
  fm = jnp.where(maskf > 0.5, feat, -jnp.inf)
  g = tr // KP
  fg = jnp.max(fm.reshape(g, KP, feat.shape[1]), axis=1)
  pf = jnp.broadcast_to(fg[:, None, :], (g, KP, feat.shape[1]))
  pf = pf.reshape(tr, feat.shape[1])
  local = loc_ref[...]
  qb = _q_expand(q_ref[...], tr)
  kd = jnp.sqrt(jnp.sum(local * local, axis=1, keepdims=True))
  wa = wa_ref[...]
  y = (_dotT(local, wa[:, 0:3]) + _dotT(qb, wa[:, 3:6])
       + _dotT(kd, wa[:, 6:7]) + _dotT(feat, wa[:, 7:135])
       + _dotT(pf, wa[:, 135:263]) + b0_ref[...])
  out_ref[...] = y
  _acc_stats(st_ref, y, maskf, pl.program_id(0))


def _stageD(z3, st3, g3, beta3, w3, b3, local, q_rows_flat, maskf,
            wa, b0, n_real):
  R, ci = z3.shape
  co = wa.shape[0]
  TR = 4096
  TQ = TR // KP
  body = functools.partial(_stageD_body, n_real=n_real)
  return pl.pallas_call(
      body,
      grid=(R // TR,),
      in_specs=[
          pl.BlockSpec((TR, ci), lambda i: (i, 0)),
          pl.BlockSpec((2, ci), lambda i: (0, 0)),
          pl.BlockSpec((ci,), lambda i: (0,)),
          pl.BlockSpec((ci,), lambda i: (0,)),
          pl.BlockSpec(w3.shape, lambda i: (0, 0)),
          pl.BlockSpec((w3.shape[0],), lambda i: (0,)),
          pl.BlockSpec((TR, 3), lambda i: (i, 0)),
          pl.BlockSpec((TQ, 3), lambda i: (i, 0)),
          pl.BlockSpec((TR, 1), lambda i: (i, 0)),
          pl.BlockSpec(wa.shape, lambda i: (0, 0)),
          pl.BlockSpec((co,), lambda i: (0,)),
      ],
      out_specs=[
          pl.BlockSpec((TR, co), lambda i: (i, 0)),
          pl.BlockSpec((2, co), lambda i: (0, 0)),
      ],
      out_shape=[
          jax.ShapeDtypeStruct((R, co), jnp.float32),
          jax.ShapeDtypeStruct((2, co), jnp.float32),
      ],
  )(z3, st3, g3, beta3, w3, b3, local, q_rows_flat, maskf, wa, b0)


def _stageG_body(z_ref, st_in_ref, g_ref, beta_ref, w_ref, b_ref, loc_ref,
                 mask_ref, out_ref, *, n_real):
  tr = z_ref.shape[0]
  a, c = _bn_aff(st_in_ref, g_ref, beta_ref, n_real)
  x = z_ref[...] * a + c
  x = jnp.where(x >= 0, x, 0.2 * x)
  wlog = _dot(x, w_ref[...]) + b_ref[...]
  g = tr // KP
  w3 = wlog.reshape(g, KP, 1)
  mask = mask_ref[...].reshape(g, KP, 1) > 0.5
  mx = jnp.max(jnp.where(mask, w3, -jnp.inf), axis=1, keepdims=True)
  e = jnp.where(mask, jnp.exp(w3 - mx), 0.0)
  s = jnp.sum(e, axis=1, keepdims=True)
  w = e / s
  loc3 = loc_ref[...].reshape(g, KP, 3)
  vec = jnp.sum(w * loc3, axis=1)
  out_ref[...] = jnp.sqrt(jnp.sum(vec * vec, axis=1, keepdims=True))


def _stageG(z, st_in, g, beta, w, b, local, maskf, n_real):
  R, ci = z.shape
  TR = 8192
  G = TR // KP
  body = functools.partial(_stageG_body, n_real=n_real)
  return pl.pallas_call(
      body,
      grid=(R // TR,),
      in_specs=[
          pl.BlockSpec((TR, ci), lambda i: (i, 0)),
          pl.BlockSpec((2, ci), lambda i: (0, 0)),
          pl.BlockSpec((ci,), lambda i: (0,)),
          pl.BlockSpec((ci,), lambda i: (0,)),
          pl.BlockSpec((ci, 1), lambda i: (0, 0)),
          pl.BlockSpec((1, 1), lambda i: (0, 0)),
          pl.BlockSpec((TR, 3), lambda i: (i, 0)),
          pl.BlockSpec((TR, 1), lambda i: (i, 0)),
      ],
      out_specs=pl.BlockSpec((G, 1), lambda i: (i, 0)),
      out_shape=jax.ShapeDtypeStruct((R // KP, 1), jnp.float32),
  )(z, st_in, g, beta, w, b, local, maskf)


def kernel(input_pcd, query_points, params):
  B, N, _ = input_pcd.shape
  M = query_points.shape[2]
  R = B * M * KP
  n_real = float(B * M * KNN)

  q_rows = jnp.transpose(query_points, (0, 2, 1))
  p_t = jnp.transpose(input_pcd, (0, 2, 1))
  q_flat = q_rows.reshape(B * M, 3)

  idx = _knn_topk(q_rows, p_t)
  idxg = idx.reshape(R)

  pf = input_pcd.reshape(B * N, 3)
  gx, gy, gz = _gather_sc(pf[:, 0], pf[:, 1], pf[:, 2], idxg)
  knn_rows = jnp.stack([gx, gy, gz], axis=1)

  maskbm = _rank_topk(gx.reshape(B * M, KP), gy.reshape(B * M, KP),
                      gz.reshape(B * M, KP), q_flat)
  maskf = maskbm.reshape(R, 1)

  p = params
  local, z1, st1 = _stageA(knn_rows, q_flat, maskf, p['patch_W0'],
                           p['patch_b0'])

  z2, st2 = _bn_mm(z1, st1, p['patch_g0'], p['patch_beta0'],
                   p['patch_W1'], p['patch_b1'], maskf, n_real)
  z3, st3 = _bn_mm(z2, st2, p['patch_g1'], p['patch_beta1'],
                   p['patch_W2'], p['patch_b2'], maskf, n_real)

  z1a, st1a = _stageD(
      z3, st3, p['patch_g2'], p['patch_beta2'],
      p['patch_W3'], p['patch_b3'], local, q_flat, maskf,
      p['attn_W0'], p['attn_b0'], n_real)

  z2a, st2a = _bn_mm(z1a, st1a, p['attn_g0'], p['attn_beta0'],
                     p['attn_W1'], p['attn_b1'], maskf, n_real)
  z3a, st3a = _bn_mm(z2a, st2a, p['attn_g1'], p['attn_beta1'],
                     p['attn_W2'], p['attn_b2'], maskf, n_real)

  udf = _stageG(z3a, st3a, p['attn_g2'], p['attn_beta2'],
                jnp.transpose(p['attn_W3']), p['attn_b3'][None, :],
                local, maskf, n_real)
  return udf.reshape(B, M)
